# baseline (speedup 1.0000x reference)
import jax
import jax.numpy as jnp
import numpy as np
from jax.experimental import pallas as pl

_B = 16384
_F = 128
_H = 256
_C = 91
_K = 45
_FILL = 0.05
_BM = 2048


def _fused_kernel(x_ref, w1_ref, b1_ref, w2_ref, b2_ref, w3_ref, b3_ref,
                  w4_ref, b4_ref, w4r_ref, b4t_ref, ones_ref, slt_ref,
                  tc_ref, logits_ref, rtt_ref):
    x = x_ref[...]
    h = jnp.maximum(jnp.dot(x, w1_ref[...], preferred_element_type=jnp.float32)
                    + b1_ref[...], 0.0)
    h = jnp.maximum(jnp.dot(h, w2_ref[...], preferred_element_type=jnp.float32)
                    + b2_ref[...], 0.0)
    h = jnp.maximum(jnp.dot(h, w3_ref[...], preferred_element_type=jnp.float32)
                    + b3_ref[...], 0.0)
    logits_ref[...] = jnp.dot(h, w4_ref[...],
                              preferred_element_type=jnp.float32) + b4_ref[...]

    lgt = jax.lax.dot_general(w4r_ref[...], h, (((1,), (1,)), ((), ())),
                              preferred_element_type=jnp.float32) + b4t_ref[...]

    si = jax.lax.bitcast_convert_type(lgt, jnp.int32)
    sm = jnp.where(si < 0, si ^ jnp.int32(0x7FFFFFFF), si)

    hi = (sm >> 16).astype(jnp.int16)
    lo = ((sm & jnp.int32(0xFFFF)) - 32768).astype(jnp.int16)

    ones_row = ones_ref[...]
    ones_bf = ones_row.astype(jnp.bfloat16)
    one_bf = jnp.bfloat16(1.0)
    zero_bf = jnp.bfloat16(0.0)

    def count_lt16(vals16, c16):
        cmpb = jnp.where(vals16 < c16, one_bf, zero_bf)
        return jnp.dot(ones_bf, cmpb, preferred_element_type=jnp.float32)

    def descend16(vals16, kvec):
        p = jnp.full((1, _BM), -32768, dtype=jnp.int32)
        for b in range(15, -1, -1):
            c = p + (1 << b)
            cnt = count_lt16(vals16, c.astype(jnp.int16))
            p = jnp.where(cnt >= kvec, p, c)
        return p

    k1 = jnp.full((1, _BM), float(_K), dtype=jnp.float32)
    p1 = descend16(hi, k1)
    p1_16 = p1.astype(jnp.int16)
    m1 = count_lt16(hi, p1_16)
    eligible = hi == p1_16
    val2 = jnp.where(eligible, lo, jnp.int16(32767))
    p2 = descend16(val2, k1 - m1)
    p = (p1 << 16) + (p2 + 32768)

    lt = sm < p
    ltf = jnp.where(lt, 1.0, 0.0)
    m = jnp.dot(ones_row, ltf, preferred_element_type=jnp.float32)
    eq = sm == p
    eqf = jnp.where(eq, 1.0, 0.0)
    pe = jnp.dot(slt_ref[...], eqf, preferred_element_type=jnp.float32)
    sel = lt | (eq & (pe < (float(_K) - m)))
    maskedt = jnp.where(sel, _FILL, lgt)
    rtt_ref[...] = jnp.dot(tc_ref[...], maskedt,
                           preferred_element_type=jnp.float32)


@jax.jit
def _run(x, w1t, b1, w2t, b2, w3t, b3, w4t, b4, w4r, b4t, ones_row, slt, tc):
    grid = (_B // _BM,)
    return pl.pallas_call(
        _fused_kernel,
        grid=grid,
        in_specs=[
            pl.BlockSpec((_BM, _F), lambda i: (i, 0)),
            pl.BlockSpec((_F, _H), lambda i: (0, 0)),
            pl.BlockSpec((1, _H), lambda i: (0, 0)),
            pl.BlockSpec((_H, _H), lambda i: (0, 0)),
            pl.BlockSpec((1, _H), lambda i: (0, 0)),
            pl.BlockSpec((_H, _H), lambda i: (0, 0)),
            pl.BlockSpec((1, _H), lambda i: (0, 0)),
            pl.BlockSpec((_H, _C), lambda i: (0, 0)),
            pl.BlockSpec((1, _C), lambda i: (0, 0)),
            pl.BlockSpec((_C, _H), lambda i: (0, 0)),
            pl.BlockSpec((_C, 1), lambda i: (0, 0)),
            pl.BlockSpec((1, _C), lambda i: (0, 0)),
            pl.BlockSpec((_C, _C), lambda i: (0, 0)),
            pl.BlockSpec((2, _C), lambda i: (0, 0)),
        ],
        out_specs=[
            pl.BlockSpec((_BM, _C), lambda i: (i, 0)),
            pl.BlockSpec((2, _BM), lambda i: (0, i)),
        ],
        out_shape=[
            jax.ShapeDtypeStruct((_B, _C), jnp.float32),
            jax.ShapeDtypeStruct((2, _B), jnp.float32),
        ],
    )(x, w1t, b1, w2t, b2, w3t, b3, w4t, b4, w4r, b4t, ones_row, slt, tc)


def kernel(x_vector, W1, b1, W2, b2, W3, b3, W4, b4, bn_gamma, bn_beta,
           bn_mean, bn_var, topic_course):
    eps = 1e-5
    scale = bn_gamma * jax.lax.rsqrt(bn_var + eps)
    shift = bn_beta - bn_mean * scale
    w1t = (W1 * scale[:, None]).T
    b1f = (b1 * scale + shift)[None, :]
    w2t = (W2 * scale[:, None]).T
    b2f = (b2 * scale + shift)[None, :]
    w3t = (W3 * scale[:, None]).T
    b3f = (b3 * scale + shift)[None, :]
    w4t = W4.T
    b4f = b4[None, :]
    b4t = b4[:, None]
    ones_row = jnp.ones((1, _C), dtype=jnp.float32)
    slt = jnp.asarray(np.tril(np.ones((_C, _C), dtype=np.float32), k=-1))
    logits, rtt = _run(x_vector, w1t, b1f, w2t, b2f, w3t, b3f, w4t, b4f,
                       W4, b4t, ones_row, slt, topic_course)
    return (logits, rtt.T)


# speedup vs baseline: 1.3989x; 1.3989x over previous
import jax
import jax.numpy as jnp
import numpy as np
from jax.experimental import pallas as pl

_B = 16384
_F = 128
_H = 256
_C = 91
_K = 45
_FILL = 0.05
_BM = 8192


def _fused_kernel(x_ref, w1_ref, b1_ref, w2_ref, b2_ref, w3_ref, b3_ref,
                  w4_ref, b4_ref, w4r_ref, b4t_ref, ones_ref, slt_ref,
                  tc_ref, logits_ref, rtt_ref):
    x = x_ref[...]
    h = jnp.maximum(jnp.dot(x, w1_ref[...], preferred_element_type=jnp.float32)
                    + b1_ref[...], 0.0)
    h = jnp.maximum(jnp.dot(h, w2_ref[...], preferred_element_type=jnp.float32)
                    + b2_ref[...], 0.0)
    h = jnp.maximum(jnp.dot(h, w3_ref[...], preferred_element_type=jnp.float32)
                    + b3_ref[...], 0.0)
    logits_ref[...] = jnp.dot(h, w4_ref[...],
                              preferred_element_type=jnp.float32) + b4_ref[...]

    lgt = jax.lax.dot_general(w4r_ref[...], h, (((1,), (1,)), ((), ())),
                              preferred_element_type=jnp.float32) + b4t_ref[...]

    si = jax.lax.bitcast_convert_type(lgt, jnp.int32)
    sm = jnp.where(si < 0, si ^ jnp.int32(0x7FFFFFFF), si)

    hi = (sm >> 16).astype(jnp.int16)
    lo = ((sm & jnp.int32(0xFFFF)) - 32768).astype(jnp.int16)

    ones_row = ones_ref[...]
    ones_bf = ones_row.astype(jnp.bfloat16)
    one_bf = jnp.bfloat16(1.0)
    zero_bf = jnp.bfloat16(0.0)

    def count_lt16(vals16, c16):
        cmpb = jnp.where(vals16 < c16, one_bf, zero_bf)
        return jnp.dot(ones_bf, cmpb, preferred_element_type=jnp.float32)

    def descend16(vals16, kvec):
        p = jnp.full((1, _BM), -32768, dtype=jnp.int32)
        for b in range(15, -1, -1):
            c = p + (1 << b)
            cnt = count_lt16(vals16, c.astype(jnp.int16))
            p = jnp.where(cnt >= kvec, p, c)
        return p

    k1 = jnp.full((1, _BM), float(_K), dtype=jnp.float32)
    p1 = descend16(hi, k1)
    p1_16 = p1.astype(jnp.int16)
    m1 = count_lt16(hi, p1_16)
    eligible = hi == p1_16
    val2 = jnp.where(eligible, lo, jnp.int16(32767))
    p2 = descend16(val2, k1 - m1)
    p = (p1 << 16) + (p2 + 32768)

    lt = sm < p
    ltf = jnp.where(lt, 1.0, 0.0)
    m = jnp.dot(ones_row, ltf, preferred_element_type=jnp.float32)
    eq = sm == p
    eqf = jnp.where(eq, 1.0, 0.0)
    pe = jnp.dot(slt_ref[...], eqf, preferred_element_type=jnp.float32)
    sel = lt | (eq & (pe < (float(_K) - m)))
    maskedt = jnp.where(sel, _FILL, lgt)
    rtt_ref[...] = jnp.dot(tc_ref[...], maskedt,
                           preferred_element_type=jnp.float32)


@jax.jit
def _run(x, w1t, b1, w2t, b2, w3t, b3, w4t, b4, w4r, b4t, ones_row, slt, tc):
    grid = (_B // _BM,)
    return pl.pallas_call(
        _fused_kernel,
        grid=grid,
        in_specs=[
            pl.BlockSpec((_BM, _F), lambda i: (i, 0)),
            pl.BlockSpec((_F, _H), lambda i: (0, 0)),
            pl.BlockSpec((1, _H), lambda i: (0, 0)),
            pl.BlockSpec((_H, _H), lambda i: (0, 0)),
            pl.BlockSpec((1, _H), lambda i: (0, 0)),
            pl.BlockSpec((_H, _H), lambda i: (0, 0)),
            pl.BlockSpec((1, _H), lambda i: (0, 0)),
            pl.BlockSpec((_H, _C), lambda i: (0, 0)),
            pl.BlockSpec((1, _C), lambda i: (0, 0)),
            pl.BlockSpec((_C, _H), lambda i: (0, 0)),
            pl.BlockSpec((_C, 1), lambda i: (0, 0)),
            pl.BlockSpec((1, _C), lambda i: (0, 0)),
            pl.BlockSpec((_C, _C), lambda i: (0, 0)),
            pl.BlockSpec((2, _C), lambda i: (0, 0)),
        ],
        out_specs=[
            pl.BlockSpec((_BM, _C), lambda i: (i, 0)),
            pl.BlockSpec((2, _BM), lambda i: (0, i)),
        ],
        out_shape=[
            jax.ShapeDtypeStruct((_B, _C), jnp.float32),
            jax.ShapeDtypeStruct((2, _B), jnp.float32),
        ],
    )(x, w1t, b1, w2t, b2, w3t, b3, w4t, b4, w4r, b4t, ones_row, slt, tc)


def kernel(x_vector, W1, b1, W2, b2, W3, b3, W4, b4, bn_gamma, bn_beta,
           bn_mean, bn_var, topic_course):
    eps = 1e-5
    scale = bn_gamma * jax.lax.rsqrt(bn_var + eps)
    shift = bn_beta - bn_mean * scale
    w1t = (W1 * scale[:, None]).T
    b1f = (b1 * scale + shift)[None, :]
    w2t = (W2 * scale[:, None]).T
    b2f = (b2 * scale + shift)[None, :]
    w3t = (W3 * scale[:, None]).T
    b3f = (b3 * scale + shift)[None, :]
    w4t = W4.T
    b4f = b4[None, :]
    b4t = b4[:, None]
    ones_row = jnp.ones((1, _C), dtype=jnp.float32)
    slt = jnp.asarray(np.tril(np.ones((_C, _C), dtype=np.float32), k=-1))
    logits, rtt = _run(x_vector, w1t, b1f, w2t, b2f, w3t, b3f, w4t, b4f,
                       W4, b4t, ones_row, slt, topic_course)
    return (logits, rtt.T)
